# baseline (speedup 1.0000x reference)
import dataclasses
import functools

import jax
import jax.numpy as jnp
from jax import lax
from jax.experimental import pallas as pl
from jax.experimental.pallas import tpu as pltpu
from jax.experimental.pallas import tpu_sc as plsc

NC = 2
NS = 16
NW = NC * NS
LANES = 16
B = 4000


def _sc_pass(pos, edge_index, n_nodes, n_edges):
    per_w = n_edges // NW
    n_blocks = per_w // B
    n_triples = (3 * B) // (3 * LANES)

    mesh = plsc.VectorSubcoreMesh(core_axis_name="c", subcore_axis_name="s")

    cp = pltpu.CompilerParams()
    for fld, val in (("needs_layout_passes", False),
                     ("use_tc_tiling_on_sc", True)):
        if fld in pltpu.CompilerParams.__dataclass_fields__:
            cp = dataclasses.replace(cp, **{fld: val})

    @functools.partial(
        pl.kernel,
        compiler_params=cp,
        out_type=(
            jax.ShapeDtypeStruct((3 * n_edges,), jnp.float32),
            jax.ShapeDtypeStruct((NW, LANES), jnp.float32),
        ),
        mesh=mesh,
        scratch_types=[
            pltpu.VMEM_SHARED((3 * n_nodes,), jnp.float32),
            pltpu.VMEM((B,), jnp.int32),
            pltpu.VMEM((B,), jnp.int32),
            pltpu.VMEM((3 * B,), jnp.int32),
            pltpu.VMEM((3 * B,), jnp.int32),
            pltpu.VMEM((3 * B,), jnp.float32),
            pltpu.VMEM((3 * B,), jnp.float32),
            pltpu.VMEM((3 * B,), jnp.float32),
            pltpu.VMEM((LANES,), jnp.float32),
            pltpu.SemaphoreType.DMA,
            pltpu.SemaphoreType.DMA,
        ],
    )
    def body(pos_hbm, row_hbm, col_hbm, u_hbm, pmax_hbm, pos_sh, ridx, cidx,
             idx3r, idx3c, av, bv, uv, accv, sem1, sem2):
        cid = lax.axis_index("c")
        sid = lax.axis_index("s")
        wid = sid * NC + cid

        @pl.when(sid == 0)
        def _():
            pltpu.sync_copy(pos_hbm, pos_sh)

        plsc.subcore_barrier()

        l = lax.iota(jnp.int32, LANES)
        kadd = [(l + LANES * w) // 3 for w in range(3)]
        dvec = [(l + LANES * w) % 3 for w in range(3)]

        accv[...] = jnp.zeros((LANES,), jnp.float32)
        base0 = wid * per_w

        @pl.loop(0, n_blocks)
        def _(bi):
            base = base0 + bi * B
            pltpu.sync_copy(row_hbm.at[pl.ds(base, B)], ridx)
            pltpu.sync_copy(col_hbm.at[pl.ds(base, B)], cidx)

            @pl.loop(0, n_triples)
            def _(t):
                j0 = t * (3 * LANES)
                k0 = t * LANES
                for w in range(3):
                    sl = pl.ds(j0 + LANES * w, LANES)
                    kv = k0 + kadd[w]
                    rv = plsc.load_gather(ridx, [kv])
                    idx3r[sl] = rv * 3 + dvec[w]
                    cv = plsc.load_gather(cidx, [kv])
                    idx3c[sl] = cv * 3 + dvec[w]

            cp1 = pltpu.async_copy(pos_sh.at[idx3r], av, sem1)
            cp2 = pltpu.async_copy(pos_sh.at[idx3c], bv, sem2)
            cp1.wait()
            cp2.wait()

            def triple(t, acc):
                j0 = t * (3 * LANES)
                for w in range(3):
                    sl = pl.ds(j0 + LANES * w, LANES)
                    u = av[sl] - bv[sl]
                    uv[sl] = u
                    acc = jnp.maximum(acc, jnp.abs(u))
                return acc

            acc = lax.fori_loop(0, n_triples, triple,
                                jnp.zeros((LANES,), jnp.float32))
            accv[...] = jnp.maximum(accv[...], acc)
            pltpu.sync_copy(uv, u_hbm.at[pl.ds(3 * base, 3 * B)])

        pltpu.sync_copy(accv, pmax_hbm.at[wid])

    return body(pos.reshape(3 * n_nodes), edge_index[0], edge_index[1])


def _tc_rescale(u_flat, pmax, n_edges):
    rows = (3 * n_edges) // 128
    blk = 3000
    grid = rows // blk

    def body(pmax_ref, u_ref, o_ref):
        scale = 0.5 / jnp.max(pmax_ref[...])
        o_ref[...] = u_ref[...] * scale + 0.5

    out = pl.pallas_call(
        body,
        grid=(grid,),
        in_specs=[
            pl.BlockSpec((NW, LANES), lambda i: (0, 0)),
            pl.BlockSpec((blk, 128), lambda i: (i, 0)),
        ],
        out_specs=pl.BlockSpec((blk, 128), lambda i: (i, 0)),
        out_shape=jax.ShapeDtypeStruct((rows, 128), jnp.float32),
    )(pmax, u_flat.reshape(rows, 128))
    return out


def kernel(pos, edge_index):
    n_nodes = pos.shape[0]
    n_edges = edge_index.shape[1]
    u_flat, pmax = _sc_pass(pos, edge_index, n_nodes, n_edges)
    out = _tc_rescale(u_flat, pmax, n_edges)
    return out


# speedup vs baseline: 1.2016x; 1.2016x over previous
import dataclasses
import functools

import jax
import jax.numpy as jnp
from jax import lax
from jax.experimental import pallas as pl
from jax.experimental.pallas import tpu as pltpu
from jax.experimental.pallas import tpu_sc as plsc

NC = 2
NS = 16
NW = NC * NS
LANES = 16
B = 4000


def _sc_pass(pos, edge_index, n_nodes, n_edges):
    per_w = n_edges // NW
    n_blocks = per_w // B
    n_win = B // LANES

    mesh = plsc.VectorSubcoreMesh(core_axis_name="c", subcore_axis_name="s")

    cp = pltpu.CompilerParams()
    for fld, val in (("needs_layout_passes", False),
                     ("use_tc_tiling_on_sc", True)):
        if fld in pltpu.CompilerParams.__dataclass_fields__:
            cp = dataclasses.replace(cp, **{fld: val})

    @functools.partial(
        pl.kernel,
        compiler_params=cp,
        out_type=(
            jax.ShapeDtypeStruct((3 * n_edges,), jnp.float32),
            jax.ShapeDtypeStruct((NW, LANES), jnp.float32),
        ),
        mesh=mesh,
        scratch_types=[
            pltpu.VMEM_SHARED((3 * n_nodes,), jnp.float32),
            pltpu.VMEM((B,), jnp.int32),
            pltpu.VMEM((B,), jnp.int32),
            pltpu.VMEM((3 * B,), jnp.int32),
            pltpu.VMEM((3 * B,), jnp.int32),
            pltpu.VMEM((3 * B,), jnp.float32),
            pltpu.VMEM((3 * B,), jnp.float32),
            pltpu.VMEM((3 * B,), jnp.float32),
            pltpu.VMEM((LANES,), jnp.float32),
            pltpu.SemaphoreType.DMA,
            pltpu.SemaphoreType.DMA,
        ],
    )
    def body(pos_hbm, row_hbm, col_hbm, u_hbm, pmax_hbm, pos_sh, ridx, cidx,
             idx3r, idx3c, av, bv, uv, accv, sem1, sem2):
        cid = lax.axis_index("c")
        sid = lax.axis_index("s")
        wid = sid * NC + cid

        @pl.when(sid == 0)
        def _():
            pltpu.sync_copy(pos_hbm, pos_sh)

        plsc.subcore_barrier()

        accv[...] = jnp.zeros((LANES,), jnp.float32)
        base0 = wid * per_w

        @pl.loop(0, n_blocks)
        def _(bi):
            base = base0 + bi * B
            pltpu.sync_copy(row_hbm.at[pl.ds(base, B)], ridx)
            pltpu.sync_copy(col_hbm.at[pl.ds(base, B)], cidx)

            @pl.loop(0, n_win)
            def _(w):
                sl = pl.ds(w * LANES, LANES)
                rv3 = ridx[sl] * 3
                cv3 = cidx[sl] * 3
                for d in range(3):
                    sld = pl.ds(d * B + w * LANES, LANES)
                    idx3r[sld] = rv3 + d
                    idx3c[sld] = cv3 + d

            cp1 = pltpu.async_copy(pos_sh.at[idx3r], av, sem1)
            cp2 = pltpu.async_copy(pos_sh.at[idx3c], bv, sem2)
            cp1.wait()
            cp2.wait()

            def win(t, acc):
                sl = pl.ds(t * LANES, LANES)
                u = av[sl] - bv[sl]
                uv[sl] = u
                return jnp.maximum(acc, jnp.abs(u))

            acc = lax.fori_loop(0, 3 * n_win, win,
                                jnp.zeros((LANES,), jnp.float32))
            accv[...] = jnp.maximum(accv[...], acc)
            for d in range(3):
                pltpu.sync_copy(uv.at[pl.ds(d * B, B)],
                                u_hbm.at[pl.ds(d * n_edges + base, B)])

        pltpu.sync_copy(accv, pmax_hbm.at[wid])

    return body(pos.reshape(3 * n_nodes), edge_index[0], edge_index[1])


def _tc_rescale(u_flat, pmax, n_edges):
    blk = 128000
    grid = n_edges // blk

    rows_per_plane = n_edges // blk

    def body(pmax_ref, u_any, o_ref, sem):
        i = pl.program_id(0)
        for d in range(3):
            pltpu.make_async_copy(
                u_any.at[pl.ds(d * rows_per_plane + i, 1), :],
                o_ref.at[pl.ds(d, 1), :], sem).start()
        for d in range(3):
            pltpu.make_async_copy(
                u_any.at[pl.ds(d * rows_per_plane + i, 1), :],
                o_ref.at[pl.ds(d, 1), :], sem).wait()
        scale = 0.5 / jnp.max(pmax_ref[...])
        o_ref[...] = o_ref[...] * scale + 0.5

    out = pl.pallas_call(
        body,
        grid=(grid,),
        in_specs=[
            pl.BlockSpec((NW, LANES), lambda i: (0, 0)),
            pl.BlockSpec(memory_space=pl.ANY),
        ],
        out_specs=pl.BlockSpec((3, blk), lambda i: (0, i)),
        out_shape=jax.ShapeDtypeStruct((3, n_edges), jnp.float32),
        scratch_shapes=[pltpu.SemaphoreType.DMA],
    )(pmax, u_flat.reshape(3 * rows_per_plane, blk))
    return out


def kernel(pos, edge_index):
    n_nodes = pos.shape[0]
    n_edges = edge_index.shape[1]
    u_flat, pmax = _sc_pass(pos, edge_index, n_nodes, n_edges)
    out = _tc_rescale(u_flat, pmax, n_edges)
    return out.T


# speedup vs baseline: 1.2029x; 1.0011x over previous
import dataclasses
import functools

import jax
import jax.numpy as jnp
from jax import lax
from jax.experimental import pallas as pl
from jax.experimental.pallas import tpu as pltpu
from jax.experimental.pallas import tpu_sc as plsc

NC = 2
NS = 16
NW = NC * NS
LANES = 16
B = 4000


def _sc_pass(pos, edge_index, n_nodes, n_edges):
    per_w = n_edges // NW
    n_blocks = per_w // B
    n_win = B // LANES

    mesh = plsc.VectorSubcoreMesh(core_axis_name="c", subcore_axis_name="s")

    cp = pltpu.CompilerParams()
    for fld, val in (("needs_layout_passes", False),
                     ("use_tc_tiling_on_sc", True)):
        if fld in pltpu.CompilerParams.__dataclass_fields__:
            cp = dataclasses.replace(cp, **{fld: val})

    @functools.partial(
        pl.kernel,
        compiler_params=cp,
        out_type=(
            jax.ShapeDtypeStruct((3 * n_edges,), jnp.float32),
            jax.ShapeDtypeStruct((NW, LANES), jnp.float32),
        ),
        mesh=mesh,
        scratch_types=[
            pltpu.VMEM_SHARED((3 * n_nodes,), jnp.float32),
            pltpu.VMEM((B,), jnp.int32),
            pltpu.VMEM((B,), jnp.int32),
            pltpu.VMEM((3 * B,), jnp.int32),
            pltpu.VMEM((3 * B,), jnp.int32),
            pltpu.VMEM((3 * B,), jnp.float32),
            pltpu.VMEM((3 * B,), jnp.float32),
            pltpu.VMEM((3 * B,), jnp.float32),
            pltpu.VMEM((LANES,), jnp.float32),
            pltpu.SemaphoreType.DMA,
            pltpu.SemaphoreType.DMA,
        ],
    )
    def body(pos_hbm, row_hbm, col_hbm, u_hbm, pmax_hbm, pos_sh, ridx, cidx,
             idx3r, idx3c, av, bv, uv, accv, sem1, sem2):
        cid = lax.axis_index("c")
        sid = lax.axis_index("s")
        wid = sid * NC + cid

        @pl.when(sid == 0)
        def _():
            pltpu.sync_copy(pos_hbm, pos_sh)

        plsc.subcore_barrier()

        accv[...] = jnp.zeros((LANES,), jnp.float32)
        base0 = wid * per_w

        @pl.loop(0, n_blocks)
        def _(bi):
            base = base0 + bi * B
            pltpu.sync_copy(row_hbm.at[pl.ds(base, B)], ridx)
            pltpu.sync_copy(col_hbm.at[pl.ds(base, B)], cidx)

            @pl.loop(0, n_win)
            def _(w):
                sl = pl.ds(w * LANES, LANES)
                rv3 = ridx[sl] * 3
                for d in range(3):
                    idx3r[pl.ds(d * B + w * LANES, LANES)] = rv3 + d

            cp1 = pltpu.async_copy(pos_sh.at[idx3r], av, sem1)

            @pl.loop(0, n_win)
            def _(w):
                sl = pl.ds(w * LANES, LANES)
                cv3 = cidx[sl] * 3
                for d in range(3):
                    idx3c[pl.ds(d * B + w * LANES, LANES)] = cv3 + d

            cp2 = pltpu.async_copy(pos_sh.at[idx3c], bv, sem2)
            cp1.wait()
            cp2.wait()

            def win(t, acc):
                sl = pl.ds(t * LANES, LANES)
                u = av[sl] - bv[sl]
                uv[sl] = u
                return jnp.maximum(acc, jnp.abs(u))

            acc = lax.fori_loop(0, 3 * n_win, win,
                                jnp.zeros((LANES,), jnp.float32))
            accv[...] = jnp.maximum(accv[...], acc)
            for d in range(3):
                pltpu.sync_copy(uv.at[pl.ds(d * B, B)],
                                u_hbm.at[pl.ds(d * n_edges + base, B)])

        pltpu.sync_copy(accv, pmax_hbm.at[wid])

    return body(pos.reshape(3 * n_nodes), edge_index[0], edge_index[1])


def _tc_rescale(u_flat, pmax, n_edges):
    blk = 128000
    grid = n_edges // blk

    rows_per_plane = n_edges // blk

    def body(pmax_ref, u_any, o_ref, sem):
        i = pl.program_id(0)
        for d in range(3):
            pltpu.make_async_copy(
                u_any.at[pl.ds(d * rows_per_plane + i, 1), :],
                o_ref.at[pl.ds(d, 1), :], sem).start()
        for d in range(3):
            pltpu.make_async_copy(
                u_any.at[pl.ds(d * rows_per_plane + i, 1), :],
                o_ref.at[pl.ds(d, 1), :], sem).wait()
        scale = 0.5 / jnp.max(pmax_ref[...])
        o_ref[...] = o_ref[...] * scale + 0.5

    out = pl.pallas_call(
        body,
        grid=(grid,),
        in_specs=[
            pl.BlockSpec((NW, LANES), lambda i: (0, 0)),
            pl.BlockSpec(memory_space=pl.ANY),
        ],
        out_specs=pl.BlockSpec((3, blk), lambda i: (0, i)),
        out_shape=jax.ShapeDtypeStruct((3, n_edges), jnp.float32),
        scratch_shapes=[pltpu.SemaphoreType.DMA],
    )(pmax, u_flat.reshape(3 * rows_per_plane, blk))
    return out


def kernel(pos, edge_index):
    n_nodes = pos.shape[0]
    n_edges = edge_index.shape[1]
    u_flat, pmax = _sc_pass(pos, edge_index, n_nodes, n_edges)
    out = _tc_rescale(u_flat, pmax, n_edges)
    return out.T
